# baseline (speedup 1.0000x reference)
import jax
import jax.numpy as jnp
from jax import lax
from jax.experimental import pallas as pl
from jax.experimental.pallas import tpu as pltpu
from jax.experimental.pallas import tpu_sc as plsc

VOCAB = 100000
EMBED_DIM = 64
BATCH = 4096
HIST_LEN = 50
NC, NS = 2, 16
NW = NC * NS
BB = BATCH // NW
NF = EMBED_DIM // 8
WB = BATCH // 128
TP = BB + 1


def _gather_body(xT_hbm, table_hbm, out_hbm,
                 idx_v, G0, G1, T0, T1, gsem, osem):
    wid = lax.axis_index("s") * NC + lax.axis_index("c")
    pltpu.sync_copy(xT_hbm.at[:, wid], idx_v)

    iota = lax.iota(jnp.int32, 16)
    fidx = [iota + (k * 16) for k in range(4)]

    def gather(h, Gp):
        pltpu.async_copy(table_hbm.at[idx_v.at[h]], Gp, gsem)

    def drain_gather(h, Gp):
        pltpu.make_async_copy(table_hbm.at[idx_v.at[h]], Gp, gsem).wait()

    def fire_out(h, Tp):
        for F in range(NF):
            pltpu.async_copy(Tp.at[pl.ds(F * 8, 8), pl.ds(0, BB)],
                             out_hbm.at[h, F, wid], osem)

    def drain_out(h, Tp):
        for F in range(NF):
            pltpu.make_async_copy(Tp.at[pl.ds(F * 8, 8), pl.ds(0, BB)],
                                  out_hbm.at[h, F, wid], osem).wait()

    def transpose(Gp, Tp):
        @plsc.parallel_loop(0, BB, unroll=8)
        def _(b):
            bvec = jnp.full((16,), b, jnp.int32)
            for k in range(4):
                v = Gp[b, pl.ds(k * 16, 16)]
                plsc.store_scatter(Tp, [fidx[k], bvec], v)

    gather(0, G0)

    def body(t, carry):
        h0 = 2 * t
        h1 = h0 + 1

        gather(h1, G1)
        drain_gather(h0, G0)

        @pl.when(t >= 1)
        def _():
            drain_out(h0 - 2, T0)

        transpose(G0, T0)
        fire_out(h0, T0)

        @pl.when(h1 < HIST_LEN - 1)
        def _():
            gather(h1 + 1, G0)

        drain_gather(h1, G1)

        @pl.when(t >= 1)
        def _():
            drain_out(h1 - 2, T1)

        transpose(G1, T1)
        fire_out(h1, T1)
        return carry

    lax.fori_loop(0, HIST_LEN // 2, body, 0)
    drain_out(HIST_LEN - 2, T0)
    drain_out(HIST_LEN - 1, T1)


def kernel(x, table):
    xT = jnp.swapaxes(x, 0, 1).reshape(HIST_LEN, WB, BB)
    mesh = plsc.VectorSubcoreMesh(core_axis_name="c", subcore_axis_name="s")
    k = pl.kernel(
        _gather_body,
        mesh=mesh,
        out_type=jax.ShapeDtypeStruct((HIST_LEN, NF, WB, 8, 128),
                                      jnp.float32),
        scratch_types=[
            pltpu.VMEM((HIST_LEN, BB), jnp.int32),
            pltpu.VMEM((BB, 2 * EMBED_DIM), jnp.float32),
            pltpu.VMEM((BB, 2 * EMBED_DIM), jnp.float32),
            pltpu.VMEM((EMBED_DIM, TP), jnp.float32),
            pltpu.VMEM((EMBED_DIM, TP), jnp.float32),
            pltpu.SemaphoreType.DMA,
            pltpu.SemaphoreType.DMA,
        ],
        compiler_params=pltpu.CompilerParams(use_tc_tiling_on_sc=False,
                                             needs_layout_passes=False),
    )
    tpad = jnp.pad(table, ((0, 0), (0, EMBED_DIM)))
    o5 = k(xT, tpad)
    return o5.transpose(2, 4, 0, 1, 3).reshape(BATCH, HIST_LEN, EMBED_DIM)


# speedup vs baseline: 1.0489x; 1.0489x over previous
import jax
import jax.numpy as jnp
from jax import lax
from jax.experimental import pallas as pl
from jax.experimental.pallas import tpu as pltpu
from jax.experimental.pallas import tpu_sc as plsc

VOCAB = 100000
EMBED_DIM = 64
BATCH = 4096
HIST_LEN = 50
NC, NS = 2, 16
NW = NC * NS
BB = BATCH // NW
NF = EMBED_DIM // 8
WB = BATCH // 128
TP = BB + 1


def _gather_body(xT_hbm, table_hbm, out_hbm,
                 idx_v, G0, G1, T0, T1, gsem, osem):
    wid = lax.axis_index("s") * NC + lax.axis_index("c")
    pltpu.sync_copy(xT_hbm.at[:, wid], idx_v)

    iota = lax.iota(jnp.int32, 16)
    fvec = [(iota + (k * 16)) // 8 for k in range(4)]
    rvec = iota % 8

    def gather(h, Gp):
        pltpu.async_copy(table_hbm.at[idx_v.at[h]], Gp, gsem)

    def drain_gather(h, Gp):
        pltpu.make_async_copy(table_hbm.at[idx_v.at[h]], Gp, gsem).wait()

    def fire_out(h, Tp):
        pltpu.async_copy(Tp.at[:, :, pl.ds(0, BB)],
                         out_hbm.at[h, :, wid], osem)

    def drain_out(h, Tp):
        pltpu.make_async_copy(Tp.at[:, :, pl.ds(0, BB)],
                              out_hbm.at[h, :, wid], osem).wait()

    def transpose(Gp, Tp):
        @plsc.parallel_loop(0, BB, unroll=8)
        def _(b):
            bvec = jnp.full((16,), b, jnp.int32)
            for k in range(4):
                v = Gp[b, pl.ds(k * 16, 16)]
                plsc.store_scatter(Tp, [fvec[k], rvec, bvec], v)

    gather(0, G0)

    def body(t, carry):
        h0 = 2 * t
        h1 = h0 + 1

        gather(h1, G1)
        drain_gather(h0, G0)

        @pl.when(t >= 1)
        def _():
            drain_out(h0 - 2, T0)

        transpose(G0, T0)
        fire_out(h0, T0)

        @pl.when(h1 < HIST_LEN - 1)
        def _():
            gather(h1 + 1, G0)

        drain_gather(h1, G1)

        @pl.when(t >= 1)
        def _():
            drain_out(h1 - 2, T1)

        transpose(G1, T1)
        fire_out(h1, T1)
        return carry

    lax.fori_loop(0, HIST_LEN // 2, body, 0)
    drain_out(HIST_LEN - 2, T0)
    drain_out(HIST_LEN - 1, T1)


def kernel(x, table):
    xT = jnp.swapaxes(x, 0, 1).reshape(HIST_LEN, WB, BB)
    mesh = plsc.VectorSubcoreMesh(core_axis_name="c", subcore_axis_name="s")
    k = pl.kernel(
        _gather_body,
        mesh=mesh,
        out_type=jax.ShapeDtypeStruct((HIST_LEN, NF, WB, 8, 128),
                                      jnp.float32),
        scratch_types=[
            pltpu.VMEM((HIST_LEN, BB), jnp.int32),
            pltpu.VMEM((BB, EMBED_DIM), jnp.float32),
            pltpu.VMEM((BB, EMBED_DIM), jnp.float32),
            pltpu.VMEM((NF, 8, TP), jnp.float32),
            pltpu.VMEM((NF, 8, TP), jnp.float32),
            pltpu.SemaphoreType.DMA,
            pltpu.SemaphoreType.DMA,
        ],
        compiler_params=pltpu.CompilerParams(use_tc_tiling_on_sc=False,
                                             needs_layout_passes=False),
    )
    o5 = k(xT, table)
    return o5.transpose(2, 4, 0, 1, 3).reshape(BATCH, HIST_LEN, EMBED_DIM)


# speedup vs baseline: 1.1231x; 1.0707x over previous
import jax
import jax.numpy as jnp
from jax import lax
from jax.experimental import pallas as pl
from jax.experimental.pallas import tpu as pltpu
from jax.experimental.pallas import tpu_sc as plsc

VOCAB = 100000
EMBED_DIM = 64
BATCH = 4096
HIST_LEN = 50
NC, NS = 2, 16
NW = NC * NS
BB = BATCH // NW
NF = EMBED_DIM // 8
WB = BATCH // 128
TP = BB + 1


def _gather_body(xT_hbm, table_hbm, out_hbm,
                 idx_v, G0, G1, T0, T1, gsem, osem):
    wid = lax.axis_index("s") * NC + lax.axis_index("c")
    pltpu.sync_copy(xT_hbm.at[:, wid], idx_v)

    iota = lax.iota(jnp.int32, 16)
    fvec = [(iota + (k * 16)) // 8 for k in range(4)]
    rvec = iota % 8

    def gather(h, Gp):
        pltpu.async_copy(table_hbm.at[idx_v.at[h]], Gp, gsem)

    def drain_gather(h, Gp):
        pltpu.make_async_copy(table_hbm.at[idx_v.at[h]], Gp, gsem).wait()

    def fire_out(h, Tp):
        pltpu.async_copy(Tp.at[:, :, pl.ds(0, BB)],
                         out_hbm.at[h, :, wid], osem)

    def drain_out(h, Tp):
        pltpu.make_async_copy(Tp.at[:, :, pl.ds(0, BB)],
                              out_hbm.at[h, :, wid], osem).wait()

    def transpose(Gp, Tp):
        @plsc.parallel_loop(0, BB, unroll=8)
        def _(b):
            bvec = jnp.full((16,), b, jnp.int32)
            for k in range(4):
                v = Gp[b, pl.ds(k * 16, 16)]
                plsc.store_scatter(Tp, [fvec[k], rvec, bvec], v)

    gather(0, G0)

    def body(t, carry):
        h0 = 2 * t
        h1 = h0 + 1

        gather(h1, G1)
        drain_gather(h0, G0)

        @pl.when(t >= 1)
        def _():
            drain_out(h0 - 2, T0)

        transpose(G0, T0)
        fire_out(h0, T0)

        @pl.when(h1 < HIST_LEN - 1)
        def _():
            gather(h1 + 1, G0)

        drain_gather(h1, G1)

        @pl.when(t >= 1)
        def _():
            drain_out(h1 - 2, T1)

        transpose(G1, T1)
        fire_out(h1, T1)
        return carry

    lax.fori_loop(0, HIST_LEN // 2, body, 0)
    drain_out(HIST_LEN - 2, T0)
    drain_out(HIST_LEN - 1, T1)


def kernel(x, table):
    xT = jnp.swapaxes(x, 0, 1).reshape(HIST_LEN, WB, BB) * 2
    mesh = plsc.VectorSubcoreMesh(core_axis_name="c", subcore_axis_name="s")
    k = pl.kernel(
        _gather_body,
        mesh=mesh,
        out_type=jax.ShapeDtypeStruct((HIST_LEN, NF, WB, 8, 128),
                                      jnp.float32),
        scratch_types=[
            pltpu.VMEM((HIST_LEN, BB), jnp.int32),
            pltpu.VMEM((BB, EMBED_DIM), jnp.float32),
            pltpu.VMEM((BB, EMBED_DIM), jnp.float32),
            pltpu.VMEM((NF, 8, TP), jnp.float32),
            pltpu.VMEM((NF, 8, TP), jnp.float32),
            pltpu.SemaphoreType.DMA,
            pltpu.SemaphoreType.DMA,
        ],
        compiler_params=pltpu.CompilerParams(use_tc_tiling_on_sc=False,
                                             needs_layout_passes=False),
    )
    tpad = jnp.pad(table, ((0, 0), (0, EMBED_DIM))).reshape(2 * VOCAB,
                                                            EMBED_DIM)
    o5 = k(xT, tpad)
    return o5.transpose(2, 4, 0, 1, 3).reshape(BATCH, HIST_LEN, EMBED_DIM)


# speedup vs baseline: 1.2170x; 1.0836x over previous
import jax
import jax.numpy as jnp
from jax import lax
from jax.experimental import pallas as pl
from jax.experimental.pallas import tpu as pltpu
from jax.experimental.pallas import tpu_sc as plsc

VOCAB = 100000
EMBED_DIM = 64
BATCH = 4096
HIST_LEN = 50
NC, NS = 2, 16
NW = NC * NS
BB = BATCH // NW
NF = EMBED_DIM // 8
WB = BATCH // 128
TP = BB + 1


def _gather_body(xT_hbm, table_hbm, out_hbm, idx_v,
                 G0, G1, G2, G3, T0, T1, T2, T3,
                 g0, g1, g2, g3, o0, o1, o2, o3):
    wid = lax.axis_index("s") * NC + lax.axis_index("c")
    pltpu.sync_copy(xT_hbm.at[:, wid], idx_v)

    G = [G0, G1, G2, G3]
    T = [T0, T1, T2, T3]
    gs = [g0, g1, g2, g3]
    os_ = [o0, o1, o2, o3]

    iota = lax.iota(jnp.int32, 16)
    fvec = [(iota + (k * 16)) // 8 for k in range(4)]
    rvec = iota % 8

    def gather(h, j):
        pltpu.async_copy(table_hbm.at[idx_v.at[h]], G[j], gs[j])

    def drain_gather(h, j):
        pltpu.make_async_copy(table_hbm.at[idx_v.at[h]], G[j], gs[j]).wait()

    def fire_out(h, j):
        pltpu.async_copy(T[j].at[:, :, pl.ds(0, BB)],
                         out_hbm.at[h, :, wid], os_[j])

    def drain_out(h, j):
        pltpu.make_async_copy(T[j].at[:, :, pl.ds(0, BB)],
                              out_hbm.at[h, :, wid], os_[j]).wait()

    def transpose(j):
        Gp, Tp = G[j], T[j]

        @plsc.parallel_loop(0, BB, unroll=8)
        def _(b):
            bvec = jnp.full((16,), b, jnp.int32)
            for k in range(4):
                v = Gp[b, pl.ds(k * 16, 16)]
                plsc.store_scatter(Tp, [fvec[k], rvec, bvec], v)

    for j in range(4):
        gather(j, j)

    def body(s, carry):
        for j in range(4):
            h = 4 * s + j
            drain_gather(h, j)

            @pl.when(s >= 1)
            def _():
                drain_out(h - 4, j)

            transpose(j)
            fire_out(h, j)

            @pl.when(h + 4 < HIST_LEN)
            def _():
                gather(h + 4, j)
        return carry

    lax.fori_loop(0, (HIST_LEN - 2) // 4, body, 0)
    for j in range(2):
        h = HIST_LEN - 2 + j
        drain_gather(h, j)
        drain_out(h - 4, j)
        transpose(j)
        fire_out(h, j)
    drain_out(HIST_LEN - 4, 2)
    drain_out(HIST_LEN - 3, 3)
    drain_out(HIST_LEN - 2, 0)
    drain_out(HIST_LEN - 1, 1)


def kernel(x, table):
    xT = jnp.swapaxes(x, 0, 1).reshape(HIST_LEN, WB, BB) * 2
    mesh = plsc.VectorSubcoreMesh(core_axis_name="c", subcore_axis_name="s")
    k = pl.kernel(
        _gather_body,
        mesh=mesh,
        out_type=jax.ShapeDtypeStruct((HIST_LEN, NF, WB, 8, 128),
                                      jnp.float32),
        scratch_types=(
            [pltpu.VMEM((HIST_LEN, BB), jnp.int32)]
            + [pltpu.VMEM((BB, EMBED_DIM), jnp.float32)] * 4
            + [pltpu.VMEM((NF, 8, TP), jnp.float32)] * 4
            + [pltpu.SemaphoreType.DMA] * 8
        ),
        compiler_params=pltpu.CompilerParams(use_tc_tiling_on_sc=False,
                                             needs_layout_passes=False),
    )
    tpad = jnp.pad(table, ((0, 0), (0, EMBED_DIM))).reshape(2 * VOCAB,
                                                            EMBED_DIM)
    o5 = k(xT, tpad)
    return o5.transpose(2, 4, 0, 1, 3).reshape(BATCH, HIST_LEN, EMBED_DIM)


# speedup vs baseline: 1.2730x; 1.0460x over previous
import jax
import jax.numpy as jnp
from jax import lax
from jax.experimental import pallas as pl
from jax.experimental.pallas import tpu as pltpu
from jax.experimental.pallas import tpu_sc as plsc

VOCAB = 100000
EMBED_DIM = 64
BATCH = 4096
HIST_LEN = 50
NC, NS = 2, 16
NW = NC * NS
BB = BATCH // NW
NF = EMBED_DIM // 8
WB = BATCH // 128
TP = BB + 1


def _gather_body(xT_hbm, table_hbm, out_hbm, idx_v,
                 G0, G1, G2, G3, T0, T1, T2, T3,
                 g0, g1, g2, g3, o0, o1, o2, o3):
    wid = lax.axis_index("s") * NC + lax.axis_index("c")
    pltpu.sync_copy(xT_hbm.at[:, wid], idx_v)

    G = [G0, G1, G2, G3]
    T = [T0, T1, T2, T3]
    gs = [g0, g1, g2, g3]
    os_ = [o0, o1, o2, o3]

    iota = lax.iota(jnp.int32, 16)
    fvec = [(iota + (k * 16)) // 8 for k in range(4)]
    rvec = iota % 8

    def gather(h, j):
        pltpu.async_copy(table_hbm.at[idx_v.at[h]], G[j], gs[j])

    def drain_gather(h, j):
        pltpu.make_async_copy(table_hbm.at[idx_v.at[h]], G[j], gs[j]).wait()

    def fire_out(h, j):
        pltpu.async_copy(T[j].at[:, :, pl.ds(0, BB)],
                         out_hbm.at[h, :, wid], os_[j])

    def drain_out(h, j):
        pltpu.make_async_copy(T[j].at[:, :, pl.ds(0, BB)],
                              out_hbm.at[h, :, wid], os_[j]).wait()

    def transpose(j):
        Gp, Tp = G[j], T[j]

        @plsc.parallel_loop(0, BB, unroll=8)
        def _(b):
            bvec = jnp.full((16,), b, jnp.int32)
            for k in range(4):
                v = Gp[b, pl.ds(k * 16, 16)]
                plsc.store_scatter(Tp, [fvec[k], rvec, bvec], v)

    for j in range(4):
        gather(j, j)

    def body(s, carry):
        for j in range(4):
            h = 4 * s + j
            drain_gather(h, j)

            @pl.when(s >= 1)
            def _():
                drain_out(h - 4, j)

            transpose(j)
            fire_out(h, j)

            @pl.when(h + 4 < HIST_LEN)
            def _():
                gather(h + 4, j)
        return carry

    lax.fori_loop(0, (HIST_LEN - 2) // 4, body, 0)
    for j in range(2):
        h = HIST_LEN - 2 + j
        drain_gather(h, j)
        drain_out(h - 4, j)
        transpose(j)
        fire_out(h, j)
    drain_out(HIST_LEN - 4, 2)
    drain_out(HIST_LEN - 3, 3)
    drain_out(HIST_LEN - 2, 0)
    drain_out(HIST_LEN - 1, 1)


VB = 2048
NVB = -(-VOCAB // VB)


def _pad_body(tT_ref, out_ref):
    t = tT_ref[...].T
    out_ref[...] = jnp.concatenate([t, jnp.zeros_like(t)], axis=1)


def kernel(x, table):
    xT = jnp.swapaxes(x, 0, 1).reshape(HIST_LEN, WB, BB) * 2
    mesh = plsc.VectorSubcoreMesh(core_axis_name="c", subcore_axis_name="s")
    k = pl.kernel(
        _gather_body,
        mesh=mesh,
        out_type=jax.ShapeDtypeStruct((HIST_LEN, NF, WB, 8, 128),
                                      jnp.float32),
        scratch_types=(
            [pltpu.VMEM((HIST_LEN, BB), jnp.int32)]
            + [pltpu.VMEM((BB, EMBED_DIM), jnp.float32)] * 4
            + [pltpu.VMEM((NF, 8, TP), jnp.float32)] * 4
            + [pltpu.SemaphoreType.DMA] * 8
        ),
        compiler_params=pltpu.CompilerParams(use_tc_tiling_on_sc=False,
                                             needs_layout_passes=False),
    )
    tT = jnp.swapaxes(table, 0, 1)
    tpad = pl.pallas_call(
        _pad_body,
        grid=(NVB,),
        in_specs=[pl.BlockSpec((EMBED_DIM, VB), lambda i: (0, i))],
        out_specs=pl.BlockSpec((VB, 2 * EMBED_DIM), lambda i: (i, 0)),
        out_shape=jax.ShapeDtypeStruct((VOCAB, 2 * EMBED_DIM), jnp.float32),
    )(tT).reshape(2 * VOCAB, EMBED_DIM)
    o5 = k(xT, tpad)
    return o5.transpose(2, 4, 0, 1, 3).reshape(BATCH, HIST_LEN, EMBED_DIM)


# speedup vs baseline: 1.4366x; 1.1286x over previous
import jax
import jax.numpy as jnp
from jax import lax
from jax.experimental import pallas as pl
from jax.experimental.pallas import tpu as pltpu
from jax.experimental.pallas import tpu_sc as plsc

VOCAB = 100000
EMBED_DIM = 64
BATCH = 4096
HIST_LEN = 50
NC, NS = 2, 16
NW = NC * NS
BB = BATCH // NW
NF = EMBED_DIM // 8
WB = BATCH // 128
TP = BB + 1


def _gather_body(xT_hbm, table_hbm, out_hbm, idx_v,
                 G0, G1, G2, G3, T0, T1, T2, T3,
                 g0, g1, g2, g3, o0, o1, o2, o3):
    wid = lax.axis_index("s") * NC + lax.axis_index("c")
    pltpu.sync_copy(xT_hbm.at[:, wid], idx_v)

    G = [G0, G1, G2, G3]
    T = [T0, T1, T2, T3]
    gs = [g0, g1, g2, g3]
    os_ = [o0, o1, o2, o3]

    iota = lax.iota(jnp.int32, 16)
    fvec = [(iota + (k * 16)) // 8 for k in range(4)]
    rvec = iota % 8

    def gather(h, j):
        pltpu.async_copy(table_hbm.at[idx_v.at[h]], G[j], gs[j])

    def drain_gather(h, j):
        pltpu.make_async_copy(table_hbm.at[idx_v.at[h]], G[j], gs[j]).wait()

    def fire_out(h, j):
        pltpu.async_copy(T[j].at[:, :, pl.ds(0, BB)],
                         out_hbm.at[h, :, wid], os_[j])

    def drain_out(h, j):
        pltpu.make_async_copy(T[j].at[:, :, pl.ds(0, BB)],
                              out_hbm.at[h, :, wid], os_[j]).wait()

    def transpose(j):
        Gp, Tp = G[j], T[j]

        @plsc.parallel_loop(0, BB, unroll=8)
        def _(b):
            bvec = jnp.full((16,), b, jnp.int32)
            for k in range(4):
                v = Gp[b, pl.ds(k * 16, 16)]
                plsc.store_scatter(Tp, [fvec[k], rvec, bvec], v)

    for j in range(4):
        gather(j, j)

    def body(s, carry):
        for j in range(4):
            h = 4 * s + j
            drain_gather(h, j)

            @pl.when(s >= 1)
            def _():
                drain_out(h - 4, j)

            transpose(j)
            fire_out(h, j)

            @pl.when(h + 4 < HIST_LEN)
            def _():
                gather(h + 4, j)
        return carry

    lax.fori_loop(0, (HIST_LEN - 2) // 4, body, 0)
    for j in range(2):
        h = HIST_LEN - 2 + j
        drain_gather(h, j)
        drain_out(h - 4, j)
        transpose(j)
        fire_out(h, j)
    drain_out(HIST_LEN - 4, 2)
    drain_out(HIST_LEN - 3, 3)
    drain_out(HIST_LEN - 2, 0)
    drain_out(HIST_LEN - 1, 1)


VB = 4096
NVB = -(-VOCAB // VB)


def _pad_body(tT_ref, out_ref):
    t = tT_ref[...].T
    out_ref[...] = jnp.concatenate([t, jnp.zeros_like(t)], axis=1)


def kernel(x, table):
    xT = jnp.swapaxes(x, 0, 1).reshape(HIST_LEN, WB, BB) * 2
    mesh = plsc.VectorSubcoreMesh(core_axis_name="c", subcore_axis_name="s")
    k = pl.kernel(
        _gather_body,
        mesh=mesh,
        out_type=jax.ShapeDtypeStruct((HIST_LEN, NF, WB, 8, 128),
                                      jnp.float32),
        scratch_types=(
            [pltpu.VMEM((HIST_LEN, BB), jnp.int32)]
            + [pltpu.VMEM((BB, EMBED_DIM), jnp.float32)] * 4
            + [pltpu.VMEM((NF, 8, TP), jnp.float32)] * 4
            + [pltpu.SemaphoreType.DMA] * 8
        ),
        compiler_params=pltpu.CompilerParams(use_tc_tiling_on_sc=False,
                                             needs_layout_passes=False),
    )
    tT = jnp.swapaxes(table, 0, 1)
    tpad = pl.pallas_call(
        _pad_body,
        grid=(NVB,),
        in_specs=[pl.BlockSpec((EMBED_DIM, VB), lambda i: (0, i))],
        out_specs=pl.BlockSpec((VB, 2 * EMBED_DIM), lambda i: (i, 0)),
        out_shape=jax.ShapeDtypeStruct((VOCAB, 2 * EMBED_DIM), jnp.float32),
    )(tT).reshape(2 * VOCAB, EMBED_DIM)
    o5 = k(xT, tpad)
    return o5.transpose(2, 4, 0, 1, 3).reshape(BATCH, HIST_LEN, EMBED_DIM)


# speedup vs baseline: 1.5644x; 1.0889x over previous
import jax
import jax.numpy as jnp
from jax import lax
from jax.experimental import pallas as pl
from jax.experimental.pallas import tpu as pltpu
from jax.experimental.pallas import tpu_sc as plsc

VOCAB = 100000
EMBED_DIM = 64
BATCH = 4096
HIST_LEN = 50
NC, NS = 2, 16
NW = NC * NS
BB = BATCH // NW
NF = EMBED_DIM // 8
WB = BATCH // 128
TP = BB + 1


def _gather_body(xT_hbm, table_hbm, out_hbm, idx_v,
                 G0, G1, G2, G3, T0, T1, T2, T3,
                 g0, g1, g2, g3, o0, o1, o2, o3):
    wid = lax.axis_index("s") * NC + lax.axis_index("c")
    pltpu.sync_copy(xT_hbm.at[:, wid], idx_v)

    G = [G0, G1, G2, G3]
    T = [T0, T1, T2, T3]
    gs = [g0, g1, g2, g3]
    os_ = [o0, o1, o2, o3]

    iota = lax.iota(jnp.int32, 16)
    fvec = [(iota + (k * 16)) // 8 for k in range(4)]
    rvec = iota % 8

    def gather(h, j):
        pltpu.async_copy(table_hbm.at[idx_v.at[h]], G[j], gs[j])

    def drain_gather(h, j):
        pltpu.make_async_copy(table_hbm.at[idx_v.at[h]], G[j], gs[j]).wait()

    def fire_out(h, j):
        pltpu.async_copy(T[j].at[:, :, pl.ds(0, BB)],
                         out_hbm.at[h, :, wid], os_[j])

    def drain_out(h, j):
        pltpu.make_async_copy(T[j].at[:, :, pl.ds(0, BB)],
                              out_hbm.at[h, :, wid], os_[j]).wait()

    def transpose(j):
        Gp, Tp = G[j], T[j]

        @plsc.parallel_loop(0, BB, unroll=8)
        def _(b):
            bvec = jnp.full((16,), b, jnp.int32)
            for k in range(4):
                v = Gp[b, pl.ds(k * 16, 16)]
                plsc.store_scatter(Tp, [fvec[k], rvec, bvec], v)

    for j in range(4):
        gather(j, j)

    def body(s, carry):
        for j in range(4):
            h = 4 * s + j
            drain_gather(h, j)

            @pl.when(s >= 1)
            def _():
                drain_out(h - 4, j)

            transpose(j)
            fire_out(h, j)

            @pl.when(h + 4 < HIST_LEN)
            def _():
                gather(h + 4, j)
        return carry

    lax.fori_loop(0, (HIST_LEN - 2) // 4, body, 0)
    for j in range(2):
        h = HIST_LEN - 2 + j
        drain_gather(h, j)
        drain_out(h - 4, j)
        transpose(j)
        fire_out(h, j)
    drain_out(HIST_LEN - 4, 2)
    drain_out(HIST_LEN - 3, 3)
    drain_out(HIST_LEN - 2, 0)
    drain_out(HIST_LEN - 1, 1)


VB = 8192
NVB = -(-VOCAB // VB)


def _pad_body(tT_ref, out_ref):
    t = tT_ref[...].T
    out_ref[...] = jnp.concatenate([t, jnp.zeros_like(t)], axis=1)


def kernel(x, table):
    xT = jnp.swapaxes(x, 0, 1).reshape(HIST_LEN, WB, BB) * 2
    mesh = plsc.VectorSubcoreMesh(core_axis_name="c", subcore_axis_name="s")
    k = pl.kernel(
        _gather_body,
        mesh=mesh,
        out_type=jax.ShapeDtypeStruct((HIST_LEN, NF, WB, 8, 128),
                                      jnp.float32),
        scratch_types=(
            [pltpu.VMEM((HIST_LEN, BB), jnp.int32)]
            + [pltpu.VMEM((BB, EMBED_DIM), jnp.float32)] * 4
            + [pltpu.VMEM((NF, 8, TP), jnp.float32)] * 4
            + [pltpu.SemaphoreType.DMA] * 8
        ),
        compiler_params=pltpu.CompilerParams(use_tc_tiling_on_sc=False,
                                             needs_layout_passes=False),
    )
    tT = jnp.swapaxes(table, 0, 1)
    tpad = pl.pallas_call(
        _pad_body,
        grid=(NVB,),
        in_specs=[pl.BlockSpec((EMBED_DIM, VB), lambda i: (0, i))],
        out_specs=pl.BlockSpec((VB, 2 * EMBED_DIM), lambda i: (i, 0)),
        out_shape=jax.ShapeDtypeStruct((VOCAB, 2 * EMBED_DIM), jnp.float32),
    )(tT).reshape(2 * VOCAB, EMBED_DIM)
    o5 = k(xT, tpad)
    return o5.transpose(2, 4, 0, 1, 3).reshape(BATCH, HIST_LEN, EMBED_DIM)


# speedup vs baseline: 1.5920x; 1.0176x over previous
import jax
import jax.numpy as jnp
from jax import lax
from jax.experimental import pallas as pl
from jax.experimental.pallas import tpu as pltpu
from jax.experimental.pallas import tpu_sc as plsc

VOCAB = 100000
EMBED_DIM = 64
BATCH = 4096
HIST_LEN = 50
NC, NS = 2, 16
NW = NC * NS
BB = BATCH // NW
NF = EMBED_DIM // 8
WB = BATCH // 128
TP = BB + 1


def _gather_body(xT_hbm, table_hbm, out_hbm, idx_v,
                 G0, G1, G2, G3, T0, T1, T2, T3,
                 g0, g1, g2, g3, o0, o1, o2, o3):
    wid = lax.axis_index("s") * NC + lax.axis_index("c")
    pltpu.sync_copy(xT_hbm.at[:, wid], idx_v)

    G = [G0, G1, G2, G3]
    T = [T0, T1, T2, T3]
    gs = [g0, g1, g2, g3]
    os_ = [o0, o1, o2, o3]

    iota = lax.iota(jnp.int32, 16)
    fvec = [(iota + (k * 16)) // 8 for k in range(4)]
    rvec = iota % 8

    def gather(h, j):
        pltpu.async_copy(table_hbm.at[idx_v.at[h]], G[j], gs[j])

    def drain_gather(h, j):
        pltpu.make_async_copy(table_hbm.at[idx_v.at[h]], G[j], gs[j]).wait()

    def fire_out(h, j):
        pltpu.async_copy(T[j].at[:, :, pl.ds(0, BB)],
                         out_hbm.at[h, :, wid], os_[j])

    def drain_out(h, j):
        pltpu.make_async_copy(T[j].at[:, :, pl.ds(0, BB)],
                              out_hbm.at[h, :, wid], os_[j]).wait()

    def transpose(j):
        Gp, Tp = G[j], T[j]

        @plsc.parallel_loop(0, BB, unroll=8)
        def _(b):
            bvec = jnp.full((16,), b, jnp.int32)
            for k in range(4):
                v = Gp[b, pl.ds(k * 16, 16)]
                plsc.store_scatter(Tp, [fvec[k], rvec, bvec], v)

    for j in range(4):
        gather(j, j)

    def body(s, carry):
        for j in range(4):
            h = 4 * s + j
            drain_gather(h, j)

            @pl.when(s >= 1)
            def _():
                drain_out(h - 4, j)

            transpose(j)
            fire_out(h, j)

            @pl.when(h + 4 < HIST_LEN)
            def _():
                gather(h + 4, j)
        return carry

    lax.fori_loop(0, (HIST_LEN - 2) // 4, body, 0)
    for j in range(2):
        h = HIST_LEN - 2 + j
        drain_gather(h, j)
        drain_out(h - 4, j)
        transpose(j)
        fire_out(h, j)
    drain_out(HIST_LEN - 4, 2)
    drain_out(HIST_LEN - 3, 3)
    drain_out(HIST_LEN - 2, 0)
    drain_out(HIST_LEN - 1, 1)


VB = 16384
NVB = -(-VOCAB // VB)


def _pad_body(tT_ref, out_ref):
    t = tT_ref[...].T
    out_ref[...] = jnp.concatenate([t, jnp.zeros_like(t)], axis=1)


def kernel(x, table):
    xT = jnp.swapaxes(x, 0, 1).reshape(HIST_LEN, WB, BB) * 2
    mesh = plsc.VectorSubcoreMesh(core_axis_name="c", subcore_axis_name="s")
    k = pl.kernel(
        _gather_body,
        mesh=mesh,
        out_type=jax.ShapeDtypeStruct((HIST_LEN, NF, WB, 8, 128),
                                      jnp.float32),
        scratch_types=(
            [pltpu.VMEM((HIST_LEN, BB), jnp.int32)]
            + [pltpu.VMEM((BB, EMBED_DIM), jnp.float32)] * 4
            + [pltpu.VMEM((NF, 8, TP), jnp.float32)] * 4
            + [pltpu.SemaphoreType.DMA] * 8
        ),
        compiler_params=pltpu.CompilerParams(use_tc_tiling_on_sc=False,
                                             needs_layout_passes=False),
    )
    tT = jnp.swapaxes(table, 0, 1)
    tpad = pl.pallas_call(
        _pad_body,
        grid=(NVB,),
        in_specs=[pl.BlockSpec((EMBED_DIM, VB), lambda i: (0, i))],
        out_specs=pl.BlockSpec((VB, 2 * EMBED_DIM), lambda i: (i, 0)),
        out_shape=jax.ShapeDtypeStruct((VOCAB, 2 * EMBED_DIM), jnp.float32),
    )(tT).reshape(2 * VOCAB, EMBED_DIM)
    o5 = k(xT, tpad)
    return o5.transpose(2, 4, 0, 1, 3).reshape(BATCH, HIST_LEN, EMBED_DIM)
